# Initial kernel scaffold; baseline (speedup 1.0000x reference)
import jax
import jax.numpy as jnp
from jax.experimental import pallas as pl


def kernel(data, W1, b1, W2, b2, Wg1, bg1, Wg2, bg2, Wl, bl):
    raise NotImplementedError("write your pallas kernel here")


# speedup vs baseline: 8.1466x; 8.1466x over previous
import numpy as np
import jax
import jax.numpy as jnp
from jax.experimental import pallas as pl
from jax.experimental.pallas import tpu as pltpu

_B, _L, _T, _OBS, _ACT = 8192, 5, 16, 64, 16
_BLK = 256


def _ahat_np():
    edges = [[i, i + 1] for i in range(_T - 1)] + [[1, 6], [2, 5]]
    a = np.eye(_T, dtype=np.float64)
    for s, d in edges:
        a[s, d] = 1.0
        a[d, s] = 1.0
    deg = a.sum(axis=1)
    dinv = 1.0 / np.sqrt(deg)
    return (dinv[:, None] * a * dinv[None, :]).astype(np.float32)


_AHAT = _ahat_np()
_DELTAS = []
for _d in range(-_T + 1, _T):
    _c = np.array([_AHAT[i, i - _d] if 0 <= i - _d < _T else 0.0
                   for i in range(_T)], dtype=np.float32)
    if np.any(_c != 0.0):
        _DELTAS.append((_d, _c))
_CMAT = np.stack([c for _, c in _DELTAS])[:, :, None]


def _agg(m2, cm, nb, f):
    m = m2.reshape(nb, _T, f)
    out = None
    for k, (delta, _) in enumerate(_DELTAS):
        c = cm[k].reshape(1, _T, 1)
        if delta == 0:
            s = m
        elif delta > 0:
            z = jnp.zeros((nb, delta, f), jnp.float32)
            s = jnp.concatenate([z, m[:, :_T - delta]], axis=1)
        else:
            kk = -delta
            z = jnp.zeros((nb, kk, f), jnp.float32)
            s = jnp.concatenate([m[:, kk:], z], axis=1)
        t = s * c
        out = t if out is None else out + t
    return out.reshape(nb * _T, f)


def _body(d_ref, w1_ref, b1_ref, w2_ref, b2_ref, wg1_ref, bg1_ref,
          wg2_ref, bg2_ref, wl_ref, bl_ref, cm_ref, out_ref):
    nb = _BLK
    cm = cm_ref[...]
    d = d_ref[...]
    xcat = jnp.concatenate([d[:, l] for l in range(_L)], axis=-1)
    xcat = xcat.reshape(nb * _T, _L * _OBS)

    def mm(x, w):
        return jax.lax.dot_general(
            x, w, (((1,), (0,)), ((), ())), preferred_element_type=jnp.float32)

    h = jax.nn.relu(mm(xcat, w1_ref[...]) + b1_ref[...])
    z = jax.nn.relu(mm(h, w2_ref[...]) + b2_ref[...])
    g1 = jax.nn.relu(mm(_agg(z, cm, nb, _OBS), wg1_ref[...]) + bg1_ref[...])
    g2 = jax.nn.relu(mm(_agg(g1, cm, nb, 128), wg2_ref[...]) + bg2_ref[...])
    y = jnp.tanh(mm(g2, wl_ref[...]) + bl_ref[...])
    pooled = jnp.sum(y.reshape(nb, _T, _ACT), axis=1) * (1.0 / _T)
    out_ref[...] = pooled


def kernel(data, W1, b1, W2, b2, Wg1, bg1, Wg2, bg2, Wl, bl):
    f32 = jnp.float32
    w1t = jnp.transpose(W1, (2, 1, 0))
    cols = []
    for p in range(3):
        blocks = []
        if p:
            blocks.append(jnp.zeros((64 * p, 64), f32))
        blocks += [w1t[0], w1t[1], w1t[2]]
        if 2 - p:
            blocks.append(jnp.zeros((64 * (2 - p), 64), f32))
        cols.append(jnp.concatenate(blocks, axis=0))
    w1big = jnp.concatenate(cols, axis=1)
    b1cat = jnp.concatenate([b1, b1, b1]).reshape(1, 192)
    w2cat = jnp.transpose(W2, (2, 1, 0)).reshape(192, 64)

    full = lambda *shape: pl.BlockSpec(shape, lambda i: (0,) * len(shape))
    grid = (_B // _BLK,)
    out = pl.pallas_call(
        _body,
        grid=grid,
        in_specs=[
            pl.BlockSpec((_BLK, _L, _T, _OBS), lambda i: (i, 0, 0, 0)),
            full(320, 192), full(1, 192),
            full(192, 64), full(1, 64),
            full(64, 128), full(1, 128),
            full(128, 128), full(1, 128),
            full(128, _ACT), full(1, _ACT),
            full(*_CMAT.shape),
        ],
        out_specs=pl.BlockSpec((_BLK, _ACT), lambda i: (i, 0)),
        out_shape=jax.ShapeDtypeStruct((_B, _ACT), f32),
        compiler_params=pltpu.CompilerParams(
            dimension_semantics=("arbitrary",)),
    )(data, w1big, b1cat, w2cat, b2.reshape(1, 64),
      Wg1, bg1.reshape(1, 128), Wg2, bg2.reshape(1, 128),
      Wl, bl.reshape(1, _ACT), jnp.asarray(_CMAT))
    return out


# speedup vs baseline: 8.8207x; 1.0827x over previous
import numpy as np
import jax
import jax.numpy as jnp
from jax.experimental import pallas as pl
from jax.experimental.pallas import tpu as pltpu

_B, _L, _T, _OBS, _ACT = 8192, 5, 16, 64, 16
_BLK = 256


def _ahat_np():
    edges = [[i, i + 1] for i in range(_T - 1)] + [[1, 6], [2, 5]]
    a = np.eye(_T, dtype=np.float64)
    for s, d in edges:
        a[s, d] = 1.0
        a[d, s] = 1.0
    deg = a.sum(axis=1)
    dinv = 1.0 / np.sqrt(deg)
    return (dinv[:, None] * a * dinv[None, :]).astype(np.float32)


_AHAT = _ahat_np()
_C0 = np.array([_AHAT[i, i] for i in range(_T)], dtype=np.float32)
_CP = np.array([_AHAT[i, i - 1] if i >= 1 else 0.0 for i in range(_T)],
               dtype=np.float32)
_CM = np.array([_AHAT[i, i + 1] if i + 1 < _T else 0.0 for i in range(_T)],
               dtype=np.float32)
_CR = {i: float(_AHAT[i, 7 - i]) for i in (1, 2, 5, 6)}
_CMAT = np.stack([_C0, _CP, _CM])[:, :, None]


def _agg(m2, cm, nb, f):
    m = m2.reshape(nb, _T, f)
    z1 = jnp.zeros((nb, 1, f), jnp.float32)
    dn = jnp.concatenate([z1, m[:, :_T - 1]], axis=1)
    up = jnp.concatenate([m[:, 1:], z1], axis=1)
    out = (m * cm[0].reshape(1, _T, 1)
           + dn * cm[1].reshape(1, _T, 1)
           + up * cm[2].reshape(1, _T, 1))
    p1 = out[:, 1:2] + _CR[1] * m[:, 6:7]
    p2 = out[:, 2:3] + _CR[2] * m[:, 5:6]
    p5 = out[:, 5:6] + _CR[5] * m[:, 2:3]
    p6 = out[:, 6:7] + _CR[6] * m[:, 1:2]
    out = jnp.concatenate(
        [out[:, 0:1], p1, p2, out[:, 3:5], p5, p6, out[:, 7:]], axis=1)
    return out.reshape(nb * _T, f)


def _body(d_ref, w1_ref, b1_ref, w2_ref, b2_ref, wg1_ref, bg1_ref,
          wg2_ref, bg2_ref, wl_ref, bl_ref, cm_ref, out_ref):
    nb = _BLK
    cm = cm_ref[...]
    d = d_ref[...]
    xcat = jnp.concatenate([d[:, l] for l in range(_L)], axis=-1)
    xcat = xcat.reshape(nb * _T, _L * _OBS)

    def mm(x, w):
        return jax.lax.dot_general(
            x, w, (((1,), (0,)), ((), ())), preferred_element_type=jnp.float32)

    h = jax.nn.relu(mm(xcat, w1_ref[...]) + b1_ref[...])
    z = jax.nn.relu(mm(h, w2_ref[...]) + b2_ref[...])
    g1 = jax.nn.relu(mm(_agg(z, cm, nb, _OBS), wg1_ref[...]) + bg1_ref[...])
    g2 = jax.nn.relu(mm(_agg(g1, cm, nb, 128), wg2_ref[...]) + bg2_ref[...])
    y = jnp.tanh(mm(g2, wl_ref[...]) + bl_ref[...])
    pooled = jnp.sum(y.reshape(nb, _T, _ACT), axis=1) * (1.0 / _T)
    out_ref[...] = pooled


def kernel(data, W1, b1, W2, b2, Wg1, bg1, Wg2, bg2, Wl, bl):
    f32 = jnp.float32
    w1t = jnp.transpose(W1, (2, 1, 0))
    cols = []
    for p in range(3):
        blocks = []
        if p:
            blocks.append(jnp.zeros((64 * p, 64), f32))
        blocks += [w1t[0], w1t[1], w1t[2]]
        if 2 - p:
            blocks.append(jnp.zeros((64 * (2 - p), 64), f32))
        cols.append(jnp.concatenate(blocks, axis=0))
    w1big = jnp.concatenate(cols, axis=1)
    b1cat = jnp.concatenate([b1, b1, b1]).reshape(1, 192)
    w2cat = jnp.transpose(W2, (2, 1, 0)).reshape(192, 64)

    full = lambda *shape: pl.BlockSpec(shape, lambda i: (0,) * len(shape))
    grid = (_B // _BLK,)
    out = pl.pallas_call(
        _body,
        grid=grid,
        in_specs=[
            pl.BlockSpec((_BLK, _L, _T, _OBS), lambda i: (i, 0, 0, 0)),
            full(320, 192), full(1, 192),
            full(192, 64), full(1, 64),
            full(64, 128), full(1, 128),
            full(128, 128), full(1, 128),
            full(128, _ACT), full(1, _ACT),
            full(*_CMAT.shape),
        ],
        out_specs=pl.BlockSpec((_BLK, _ACT), lambda i: (i, 0)),
        out_shape=jax.ShapeDtypeStruct((_B, _ACT), f32),
        compiler_params=pltpu.CompilerParams(
            dimension_semantics=("arbitrary",)),
    )(data, w1big, b1cat, w2cat, b2.reshape(1, 64),
      Wg1, bg1.reshape(1, 128), Wg2, bg2.reshape(1, 128),
      Wl, bl.reshape(1, _ACT), jnp.asarray(_CMAT))
    return out


# speedup vs baseline: 23.6621x; 2.6826x over previous
import numpy as np
import jax
import jax.numpy as jnp
from jax.experimental import pallas as pl
from jax.experimental.pallas import tpu as pltpu

_B, _L, _T, _OBS, _ACT = 8192, 5, 16, 64, 16
_BLK = 512


def _ahat_np():
    edges = [[i, i + 1] for i in range(_T - 1)] + [[1, 6], [2, 5]]
    a = np.eye(_T, dtype=np.float64)
    for s, d in edges:
        a[s, d] = 1.0
        a[d, s] = 1.0
    deg = a.sum(axis=1)
    dinv = 1.0 / np.sqrt(deg)
    return (dinv[:, None] * a * dinv[None, :]).astype(np.float32)


_AHAT = _ahat_np()
_NBRS = [[(j, float(_AHAT[i, j])) for j in range(_T) if _AHAT[i, j] != 0.0]
         for i in range(_T)]


def _body(d_ref, w1_ref, b1_ref, w2_ref, b2_ref, wg1_ref, bg1_ref,
          wg2_ref, bg2_ref, wl_ref, bl_ref, out_ref):
    def mm(w, x):
        return jax.lax.dot_general(
            w, x, (((1,), (0,)), ((), ())), preferred_element_type=jnp.float32)

    w1 = w1_ref[...]
    w2 = w2_ref[...]
    wg1 = wg1_ref[...]
    wg2 = wg2_ref[...]
    wl = wl_ref[...]
    b1 = b1_ref[...]
    b2 = b2_ref[...]
    bg1 = bg1_ref[...]
    bg2 = bg2_ref[...]
    bl = bl_ref[...]

    z = []
    for t in range(_T):
        acc = b1
        for l in range(_L):
            acc = acc + mm(w1[l], d_ref[l, t])
        h = jax.nn.relu(acc)
        z.append(jax.nn.relu(mm(w2, h) + b2))

    def agg(xs):
        outs = []
        for i in range(_T):
            a = None
            for j, c in _NBRS[i]:
                v = xs[j] * c
                a = v if a is None else a + v
            outs.append(a)
        return outs

    g1 = [jax.nn.relu(mm(wg1, v) + bg1) for v in agg(z)]
    g2 = [jax.nn.relu(mm(wg2, v) + bg2) for v in agg(g1)]
    pooled = None
    for t in range(_T):
        y = jnp.tanh(mm(wl, g2[t]) + bl)
        pooled = y if pooled is None else pooled + y
    out_ref[...] = pooled * (1.0 / _T)


def kernel(data, W1, b1, W2, b2, Wg1, bg1, Wg2, bg2, Wl, bl):
    f32 = jnp.float32
    dt = jnp.transpose(data, (1, 2, 3, 0))

    zero = jnp.zeros((64, 64), f32)
    w1col = jnp.stack([
        jnp.concatenate(
            [W1[:, :, l - p] if 0 <= l - p < 3 else zero for p in range(3)],
            axis=0)
        for l in range(_L)])
    w2cat = jnp.transpose(W2, (0, 2, 1)).reshape(64, 192)
    b1cat = jnp.concatenate([b1, b1, b1]).reshape(192, 1)

    full = lambda *shape: pl.BlockSpec(shape, lambda i: (0,) * len(shape))
    grid = (_B // _BLK,)
    out = pl.pallas_call(
        _body,
        grid=grid,
        in_specs=[
            pl.BlockSpec((_L, _T, _OBS, _BLK), lambda i: (0, 0, 0, i)),
            full(_L, 192, 64), full(192, 1),
            full(64, 192), full(64, 1),
            full(128, 64), full(128, 1),
            full(128, 128), full(128, 1),
            full(_ACT, 128), full(_ACT, 1),
        ],
        out_specs=pl.BlockSpec((_ACT, _BLK), lambda i: (0, i)),
        out_shape=jax.ShapeDtypeStruct((_ACT, _B), f32),
        compiler_params=pltpu.CompilerParams(
            dimension_semantics=("arbitrary",)),
    )(dt, w1col, b1cat, w2cat, b2.reshape(64, 1),
      jnp.transpose(Wg1), bg1.reshape(128, 1),
      jnp.transpose(Wg2), bg2.reshape(128, 1),
      jnp.transpose(Wl), bl.reshape(_ACT, 1))
    return jnp.transpose(out)


# speedup vs baseline: 39.4470x; 1.6671x over previous
import numpy as np
import jax
import jax.numpy as jnp
from jax.experimental import pallas as pl
from jax.experimental.pallas import tpu as pltpu

_B, _L, _T, _OBS, _ACT = 8192, 5, 16, 64, 16
_BLK = 512


def _ahat_np():
    edges = [[i, i + 1] for i in range(_T - 1)] + [[1, 6], [2, 5]]
    a = np.eye(_T, dtype=np.float64)
    for s, d in edges:
        a[s, d] = 1.0
        a[d, s] = 1.0
    deg = a.sum(axis=1)
    dinv = 1.0 / np.sqrt(deg)
    return (dinv[:, None] * a * dinv[None, :]).astype(np.float32)


_AHAT = _ahat_np()
_NBRS = [[(j, float(_AHAT[i, j])) for j in range(_T) if _AHAT[i, j] != 0.0]
         for i in range(_T)]


def _agg(x, w):
    cols = []
    for i in range(_T):
        a = None
        for j, c in _NBRS[i]:
            v = x[:, j * w:(j + 1) * w] * c
            a = v if a is None else a + v
        cols.append(a)
    return jnp.concatenate(cols, axis=1)


def _body(d_ref, w1_ref, b1_ref, w2_ref, b2_ref, wg1_ref, bg1_ref,
          wg2_ref, bg2_ref, wl_ref, bl_ref, out_ref):
    def mm(w, x):
        return jax.lax.dot_general(
            w, x, (((1,), (0,)), ((), ())), preferred_element_type=jnp.float32)

    dcat = jnp.concatenate([
        jnp.concatenate([d_ref[l, t] for t in range(_T)], axis=1)
        for l in range(_L)], axis=0)

    h = jax.nn.relu(mm(w1_ref[...], dcat) + b1_ref[...])
    z = jax.nn.relu(mm(w2_ref[...], h) + b2_ref[...])
    g1 = jax.nn.relu(mm(wg1_ref[...], _agg(z, _BLK)) + bg1_ref[...])
    g2 = jax.nn.relu(mm(wg2_ref[...], _agg(g1, _BLK)) + bg2_ref[...])
    y = jnp.tanh(mm(wl_ref[...], g2) + bl_ref[...])
    pooled = None
    for t in range(_T):
        s = y[:, t * _BLK:(t + 1) * _BLK]
        pooled = s if pooled is None else pooled + s
    out_ref[...] = pooled * (1.0 / _T)


def kernel(data, W1, b1, W2, b2, Wg1, bg1, Wg2, bg2, Wl, bl):
    f32 = jnp.float32
    dt = jnp.transpose(data, (1, 2, 3, 0))

    zero = jnp.zeros((64, 64), f32)
    w1big = jnp.concatenate([
        jnp.concatenate(
            [W1[:, :, l - p] if 0 <= l - p < 3 else zero for l in range(_L)],
            axis=1)
        for p in range(3)], axis=0)
    w2cat = jnp.transpose(W2, (0, 2, 1)).reshape(64, 192)
    b1cat = jnp.concatenate([b1, b1, b1]).reshape(192, 1)

    full = lambda *shape: pl.BlockSpec(shape, lambda i: (0,) * len(shape))
    grid = (_B // _BLK,)
    out = pl.pallas_call(
        _body,
        grid=grid,
        in_specs=[
            pl.BlockSpec((_L, _T, _OBS, _BLK), lambda i: (0, 0, 0, i)),
            full(192, 320), full(192, 1),
            full(64, 192), full(64, 1),
            full(128, 64), full(128, 1),
            full(128, 128), full(128, 1),
            full(_ACT, 128), full(_ACT, 1),
        ],
        out_specs=pl.BlockSpec((_ACT, _BLK), lambda i: (0, i)),
        out_shape=jax.ShapeDtypeStruct((_ACT, _B), f32),
        compiler_params=pltpu.CompilerParams(
            dimension_semantics=("arbitrary",)),
    )(dt, w1big, b1cat, w2cat, b2.reshape(64, 1),
      jnp.transpose(Wg1), bg1.reshape(128, 1),
      jnp.transpose(Wg2), bg2.reshape(128, 1),
      jnp.transpose(Wl), bl.reshape(_ACT, 1))
    return jnp.transpose(out)
